# baseline (speedup 1.0000x reference)
import functools

import jax
import jax.numpy as jnp
from jax import lax
from jax.experimental import pallas as pl
from jax.experimental.pallas import tpu as pltpu
from jax.experimental.pallas import tpu_sc as plsc

N = 10000
D = 128
E = 320000

NC = 2
NS = 16
NW = NC * NS
CHUNK = 128
K = 80
E_PAD = NW * CHUNK * K
NPAD = 10240
ROWS_PER_TILE = NPAD // NS
L = 16
NBUF = 1

_BLK = 2000


SB = 8
NSB = K // SB


@functools.lru_cache(maxsize=None)
def _make_agg(with_deg):
    mesh = plsc.VectorSubcoreMesh(core_axis_name="c", subcore_axis_name="s",
                                  num_cores=NC, num_subcores=NS)
    out_type = [jax.ShapeDtypeStruct((NC, NPAD, D), jnp.float32)]
    scratch = [
        pltpu.VMEM((2, SB, CHUNK), jnp.int32),
        pltpu.VMEM((2, SB, CHUNK), jnp.int32),
        pltpu.VMEM((2, CHUNK, D), jnp.float32),
        pltpu.VMEM_SHARED((NPAD, D), jnp.float32),
        pltpu.SemaphoreType.DMA,
        pltpu.SemaphoreType.DMA,
        pltpu.SemaphoreType.DMA,
        pltpu.SemaphoreType.DMA,
    ]
    if with_deg:
        out_type.append(jax.ShapeDtypeStruct((NW * NPAD,), jnp.float32))
        scratch.append(pltpu.VMEM((NPAD,), jnp.float32))

    @functools.partial(pl.kernel, out_type=out_type, mesh=mesh,
                       compiler_params=pltpu.CompilerParams(needs_layout_passes=False),
                       scratch_types=scratch)
    def agg(*refs):
        if with_deg:
            (feat_hbm, src_hbm, dst_hbm, zrows_hbm, zdeg_hbm, out_hbm, deg_hbm,
             sidx, didx, rows_v, acc_s, sem_s, sem_i, sem_g0, sem_g1, deg_v) = refs
        else:
            (feat_hbm, src_hbm, dst_hbm, zrows_hbm, out_hbm,
             sidx, didx, rows_v, acc_s, sem_s, sem_i, sem_g0, sem_g1) = refs
        sem_g = (sem_g0, sem_g1)
        c = lax.axis_index("c")
        s = lax.axis_index("s")
        w = c * NS + s
        rbase = s * ROWS_PER_TILE
        pltpu.sync_copy(zrows_hbm, acc_s.at[pl.ds(rbase, ROWS_PER_TILE)])
        if with_deg:
            pltpu.sync_copy(zdeg_hbm, deg_v)
        plsc.subcore_barrier()

        def idx_fill(sb, buf):
            base = w * K + sb * SB
            pltpu.async_copy(src_hbm.at[pl.ds(base, SB)], sidx.at[buf], sem_i)
            pltpu.async_copy(dst_hbm.at[pl.ds(base, SB)], didx.at[buf], sem_i)

        def idx_wait(sb, buf):
            base = w * K + sb * SB
            pltpu.make_async_copy(src_hbm.at[pl.ds(base, SB)], sidx.at[buf],
                                  sem_i).wait()
            pltpu.make_async_copy(dst_hbm.at[pl.ds(base, SB)], didx.at[buf],
                                  sem_i).wait()

        def drain_scatter(buf, row):
            pltpu.make_async_copy(rows_v.at[buf], acc_s.at[didx.at[0, row]],
                                  sem_s).wait()

        idx_fill(0, 0)

        def do_sb(sb, buf):
            @pl.when(sb > 0)
            def _():
                drain_scatter(0, 0)
                drain_scatter(1, 0)

            @pl.when(sb < NSB - 1)
            def _():
                idx_fill(sb + 1, 1 - buf)

            idx_wait(sb, buf)
            if with_deg:
                for r in range(SB):
                    for i in range(CHUNK // L):
                        idx = didx[buf, r, pl.ds(i * L, L)]
                        cnt, last = plsc.scan_count(idx)
                        plsc.addupdate_scatter(deg_v, [idx],
                                               cnt.astype(jnp.float32), mask=last)
            for g in range(SB // 2):
                gathers = []
                for b in range(2):
                    r = 2 * g + b
                    if g > 0:
                        drain_scatter(b, 0)
                    for h in range(2):
                        gathers.append(pltpu.async_copy(
                            feat_hbm.at[sidx.at[buf, r, pl.ds(h * (CHUNK // 2),
                                                              CHUNK // 2)]],
                            rows_v.at[b, pl.ds(h * (CHUNK // 2), CHUNK // 2)],
                            sem_g[b]))
                for b in range(2):
                    r = 2 * g + b
                    gathers[2 * b].wait()
                    gathers[2 * b + 1].wait()
                    pltpu.async_copy(rows_v.at[b], acc_s.at[didx.at[buf, r]],
                                     sem_s, add=True)

        def body(t, carry):
            do_sb(2 * t, 0)
            do_sb(2 * t + 1, 1)
            return carry

        lax.fori_loop(0, NSB // 2, body, 0)
        drain_scatter(0, 0)
        drain_scatter(1, 0)
        plsc.subcore_barrier()
        pltpu.sync_copy(acc_s.at[pl.ds(rbase, ROWS_PER_TILE)],
                        out_hbm.at[c, pl.ds(rbase, ROWS_PER_TILE)])
        if with_deg:
            pltpu.sync_copy(deg_v, deg_hbm.at[pl.ds(w * NPAD, NPAD)])

    return agg


def _l1_body(x_ref, p_ref, degp_ref, ws_ref, wn_ref, b_ref, h_ref, rdeg_ref):
    deg = jnp.sum(degp_ref[...], axis=1)
    r = (1.0 / jnp.maximum(deg, 1.0))[:, None]
    mean = (p_ref[0] + p_ref[1]) * r
    acc = jnp.dot(x_ref[...], ws_ref[...], preferred_element_type=jnp.float32)
    acc += jnp.dot(mean, wn_ref[...], preferred_element_type=jnp.float32)
    acc += b_ref[...]
    h_ref[...] = jnp.maximum(acc, 0.0)
    rdeg_ref[...] = r


def _l2_body(h_ref, p_ref, rdeg_ref, ws_ref, wn_ref, b_ref, o_ref):
    mean = (p_ref[0] + p_ref[1]) * rdeg_ref[...]
    acc = jnp.dot(h_ref[...], ws_ref[...], preferred_element_type=jnp.float32)
    acc += jnp.dot(mean, wn_ref[...], preferred_element_type=jnp.float32)
    o_ref[...] = acc + b_ref[...]


def _layer1(x, p1, degp, w_self, w_neigh, b):
    return pl.pallas_call(
        _l1_body,
        grid=(N // _BLK,),
        in_specs=[
            pl.BlockSpec((_BLK, D), lambda i: (i, 0)),
            pl.BlockSpec((NC, _BLK, D), lambda i: (0, i, 0)),
            pl.BlockSpec((_BLK, NW), lambda i: (i, 0)),
            pl.BlockSpec((D, D), lambda i: (0, 0)),
            pl.BlockSpec((D, D), lambda i: (0, 0)),
            pl.BlockSpec((1, D), lambda i: (0, 0)),
        ],
        out_specs=[
            pl.BlockSpec((_BLK, D), lambda i: (i, 0)),
            pl.BlockSpec((_BLK, 1), lambda i: (i, 0)),
        ],
        out_shape=[
            jax.ShapeDtypeStruct((N, D), jnp.float32),
            jax.ShapeDtypeStruct((N, 1), jnp.float32),
        ],
    )(x, p1, degp, w_self, w_neigh, b.reshape(1, D))


def _layer2(h, p2, rdeg, w_self, w_neigh, b):
    return pl.pallas_call(
        _l2_body,
        grid=(N // _BLK,),
        in_specs=[
            pl.BlockSpec((_BLK, D), lambda i: (i, 0)),
            pl.BlockSpec((NC, _BLK, D), lambda i: (0, i, 0)),
            pl.BlockSpec((_BLK, 1), lambda i: (i, 0)),
            pl.BlockSpec((D, D), lambda i: (0, 0)),
            pl.BlockSpec((D, D), lambda i: (0, 0)),
            pl.BlockSpec((1, D), lambda i: (0, 0)),
        ],
        out_specs=pl.BlockSpec((_BLK, D), lambda i: (i, 0)),
        out_shape=jax.ShapeDtypeStruct((N, D), jnp.float32),
    )(h, p2, rdeg, w_self, w_neigh, b.reshape(1, D))


def kernel(x, edge_index, W1_self, W1_neigh, b1, W2_self, W2_neigh, b2):
    src = edge_index[0]
    dst = edge_index[1]
    pad = E_PAD - E
    pad_src = jnp.arange(pad, dtype=jnp.int32) % N
    pad_dst = N + (jnp.arange(pad, dtype=jnp.int32) % (NPAD - N))
    src_p = jnp.concatenate([src, pad_src]).reshape(NW * K, CHUNK)
    dst_p = jnp.concatenate([dst, pad_dst]).reshape(NW * K, CHUNK)
    zrows = jnp.zeros((ROWS_PER_TILE, D), jnp.float32)
    zdeg = jnp.zeros((NPAD,), jnp.float32)

    p1, degf = _make_agg(True)(x, src_p, dst_p, zrows, zdeg)
    degp = degf.reshape(NW, NPAD).T
    h, rdeg = _layer1(x, p1, degp, W1_self, W1_neigh, b1)
    [p2] = _make_agg(False)(h, src_p, dst_p, zrows)
    return _layer2(h, p2, rdeg, W2_self, W2_neigh, b2)


# speedup vs baseline: 1.0029x; 1.0029x over previous
import functools

import jax
import jax.numpy as jnp
from jax import lax
from jax.experimental import pallas as pl
from jax.experimental.pallas import tpu as pltpu
from jax.experimental.pallas import tpu_sc as plsc

N = 10000
D = 128
E = 320000

NC = 2
NS = 16
NW = NC * NS
CHUNK = 128
K = 80
E_PAD = NW * CHUNK * K
NPAD = 10240
ROWS_PER_TILE = NPAD // NS
L = 16
NBUF = 1

_BLK = 2000


SB = 8
NSB = K // SB


@functools.lru_cache(maxsize=None)
def _make_agg(with_deg):
    mesh = plsc.VectorSubcoreMesh(core_axis_name="c", subcore_axis_name="s",
                                  num_cores=NC, num_subcores=NS)
    out_type = [jax.ShapeDtypeStruct((NC, NPAD, D), jnp.float32)]
    scratch = [
        pltpu.VMEM((2, SB, CHUNK), jnp.int32),
        pltpu.VMEM((2, SB, CHUNK), jnp.int32),
        pltpu.VMEM((2, CHUNK, D), jnp.float32),
        pltpu.VMEM_SHARED((NPAD, D), jnp.float32),
        pltpu.SemaphoreType.DMA,
        pltpu.SemaphoreType.DMA,
        pltpu.SemaphoreType.DMA,
        pltpu.SemaphoreType.DMA,
    ]
    if with_deg:
        out_type.append(jax.ShapeDtypeStruct((NW * NPAD,), jnp.float32))
        scratch.append(pltpu.VMEM((NPAD,), jnp.float32))

    @functools.partial(pl.kernel, out_type=out_type, mesh=mesh,
                       compiler_params=pltpu.CompilerParams(needs_layout_passes=False),
                       scratch_types=scratch)
    def agg(*refs):
        if with_deg:
            (feat_hbm, src_hbm, dst_hbm, zrows_hbm, zdeg_hbm, out_hbm, deg_hbm,
             sidx, didx, rows_v, acc_s, sem_s, sem_i, sem_g0, sem_g1, deg_v) = refs
        else:
            (feat_hbm, src_hbm, dst_hbm, zrows_hbm, out_hbm,
             sidx, didx, rows_v, acc_s, sem_s, sem_i, sem_g0, sem_g1) = refs
        sem_g = (sem_g0, sem_g1)
        c = lax.axis_index("c")
        s = lax.axis_index("s")
        w = c * NS + s
        rbase = s * ROWS_PER_TILE

        def idx_fill(sb, buf):
            base = w * K + sb * SB
            pltpu.async_copy(src_hbm.at[pl.ds(base, SB)], sidx.at[buf], sem_i)
            pltpu.async_copy(dst_hbm.at[pl.ds(base, SB)], didx.at[buf], sem_i)

        def idx_wait(sb, buf):
            base = w * K + sb * SB
            pltpu.make_async_copy(src_hbm.at[pl.ds(base, SB)], sidx.at[buf],
                                  sem_i).wait()
            pltpu.make_async_copy(dst_hbm.at[pl.ds(base, SB)], didx.at[buf],
                                  sem_i).wait()

        def drain_scatter(buf, row):
            pltpu.make_async_copy(rows_v.at[buf], acc_s.at[didx.at[0, row]],
                                  sem_s).wait()

        idx_fill(0, 0)
        pltpu.sync_copy(zrows_hbm, acc_s.at[pl.ds(rbase, ROWS_PER_TILE)])
        if with_deg:
            pltpu.sync_copy(zdeg_hbm, deg_v)
        plsc.subcore_barrier()

        def do_sb(sb, buf):
            @pl.when(sb > 0)
            def _():
                drain_scatter(0, 0)
                drain_scatter(1, 0)

            @pl.when(sb < NSB - 1)
            def _():
                idx_fill(sb + 1, 1 - buf)

            idx_wait(sb, buf)
            if with_deg:
                for r in range(SB):
                    for i in range(CHUNK // L):
                        idx = didx[buf, r, pl.ds(i * L, L)]
                        cnt, last = plsc.scan_count(idx)
                        plsc.addupdate_scatter(deg_v, [idx],
                                               cnt.astype(jnp.float32), mask=last)
            for g in range(SB // 2):
                gathers = []
                for b in range(2):
                    r = 2 * g + b
                    if g > 0:
                        drain_scatter(b, 0)
                    gathers.append(pltpu.async_copy(
                        feat_hbm.at[sidx.at[buf, r]], rows_v.at[b], sem_g[b]))
                for b in range(2):
                    r = 2 * g + b
                    gathers[b].wait()
                    pltpu.async_copy(rows_v.at[b], acc_s.at[didx.at[buf, r]],
                                     sem_s, add=True)

        def body(t, carry):
            do_sb(2 * t, 0)
            do_sb(2 * t + 1, 1)
            return carry

        lax.fori_loop(0, NSB // 2, body, 0)
        drain_scatter(0, 0)
        drain_scatter(1, 0)
        plsc.subcore_barrier()
        pltpu.sync_copy(acc_s.at[pl.ds(rbase, ROWS_PER_TILE)],
                        out_hbm.at[c, pl.ds(rbase, ROWS_PER_TILE)])
        if with_deg:
            pltpu.sync_copy(deg_v, deg_hbm.at[pl.ds(w * NPAD, NPAD)])

    return agg


def _l1_body(x_ref, p_ref, degp_ref, ws_ref, wn_ref, b_ref, h_ref, rdeg_ref):
    deg = jnp.sum(degp_ref[...], axis=1)
    r = (1.0 / jnp.maximum(deg, 1.0))[:, None]
    mean = (p_ref[0] + p_ref[1]) * r
    acc = jnp.dot(x_ref[...], ws_ref[...], preferred_element_type=jnp.float32)
    acc += jnp.dot(mean, wn_ref[...], preferred_element_type=jnp.float32)
    acc += b_ref[...]
    h_ref[...] = jnp.maximum(acc, 0.0)
    rdeg_ref[...] = r


def _l2_body(h_ref, p_ref, rdeg_ref, ws_ref, wn_ref, b_ref, o_ref):
    mean = (p_ref[0] + p_ref[1]) * rdeg_ref[...]
    acc = jnp.dot(h_ref[...], ws_ref[...], preferred_element_type=jnp.float32)
    acc += jnp.dot(mean, wn_ref[...], preferred_element_type=jnp.float32)
    o_ref[...] = acc + b_ref[...]


def _layer1(x, p1, degp, w_self, w_neigh, b):
    return pl.pallas_call(
        _l1_body,
        grid=(N // _BLK,),
        in_specs=[
            pl.BlockSpec((_BLK, D), lambda i: (i, 0)),
            pl.BlockSpec((NC, _BLK, D), lambda i: (0, i, 0)),
            pl.BlockSpec((_BLK, NW), lambda i: (i, 0)),
            pl.BlockSpec((D, D), lambda i: (0, 0)),
            pl.BlockSpec((D, D), lambda i: (0, 0)),
            pl.BlockSpec((1, D), lambda i: (0, 0)),
        ],
        out_specs=[
            pl.BlockSpec((_BLK, D), lambda i: (i, 0)),
            pl.BlockSpec((_BLK, 1), lambda i: (i, 0)),
        ],
        out_shape=[
            jax.ShapeDtypeStruct((N, D), jnp.float32),
            jax.ShapeDtypeStruct((N, 1), jnp.float32),
        ],
    )(x, p1, degp, w_self, w_neigh, b.reshape(1, D))


def _layer2(h, p2, rdeg, w_self, w_neigh, b):
    return pl.pallas_call(
        _l2_body,
        grid=(N // _BLK,),
        in_specs=[
            pl.BlockSpec((_BLK, D), lambda i: (i, 0)),
            pl.BlockSpec((NC, _BLK, D), lambda i: (0, i, 0)),
            pl.BlockSpec((_BLK, 1), lambda i: (i, 0)),
            pl.BlockSpec((D, D), lambda i: (0, 0)),
            pl.BlockSpec((D, D), lambda i: (0, 0)),
            pl.BlockSpec((1, D), lambda i: (0, 0)),
        ],
        out_specs=pl.BlockSpec((_BLK, D), lambda i: (i, 0)),
        out_shape=jax.ShapeDtypeStruct((N, D), jnp.float32),
    )(h, p2, rdeg, w_self, w_neigh, b.reshape(1, D))


def kernel(x, edge_index, W1_self, W1_neigh, b1, W2_self, W2_neigh, b2):
    src = edge_index[0]
    dst = edge_index[1]
    pad = E_PAD - E
    pad_src = jnp.arange(pad, dtype=jnp.int32) % N
    pad_dst = N + (jnp.arange(pad, dtype=jnp.int32) % (NPAD - N))
    src_p = jnp.concatenate([src, pad_src]).reshape(NW * K, CHUNK)
    dst_p = jnp.concatenate([dst, pad_dst]).reshape(NW * K, CHUNK)
    zrows = jnp.zeros((ROWS_PER_TILE, D), jnp.float32)
    zdeg = jnp.zeros((NPAD,), jnp.float32)

    p1, degf = _make_agg(True)(x, src_p, dst_p, zrows, zdeg)
    degp = degf.reshape(NW, NPAD).T
    h, rdeg = _layer1(x, p1, degp, W1_self, W1_neigh, b1)
    [p2] = _make_agg(False)(h, src_p, dst_p, zrows)
    return _layer2(h, p2, rdeg, W2_self, W2_neigh, b2)
